# baseline (speedup 1.0000x reference)
import functools

import jax
import jax.numpy as jnp
from jax import lax
from jax.experimental import pallas as pl
from jax.experimental.pallas import tpu as pltpu, tpu_sc as plsc


@functools.cache
def _build(V, D, B):
    info = plsc.get_sparse_core_info()
    NC, NS, L = info.num_cores, info.num_subcores, info.num_lanes
    NW = NC * NS
    assert B % (8 * NW) == 0 and D % L == 0
    b_per_w = B // NW
    n_el = b_per_w * D
    mesh = plsc.VectorSubcoreMesh(core_axis_name="c", subcore_axis_name="s")

    @functools.partial(
        pl.kernel,
        mesh=mesh,
        out_type=jax.ShapeDtypeStruct((D, B), jnp.float32),
        scratch_types=[
            pltpu.VMEM((b_per_w,), jnp.int32),
            pltpu.VMEM((n_el,), jnp.int32),
            pltpu.VMEM((n_el,), jnp.float32),
            pltpu.SemaphoreType.DMA,
            pltpu.SemaphoreType.DMA,
        ],
        compiler_params=pltpu.CompilerParams(
            disable_bounds_checks=True,
            disable_semaphore_checks=True,
        ),
    )
    def k(t_hbm, idx_hbm, out_hbm, idx_v, gidx_v, buf_v, sem, sem2):
        wid = lax.axis_index("s") * NC + lax.axis_index("c")
        base = wid * b_per_w
        pltpu.sync_copy(idx_hbm.at[pl.ds(base, b_per_w)], idx_v)

        def build_g(g, _):
            v = idx_v[pl.ds(g * L, L)]
            for j in range(D):
                gidx_v[pl.ds(j * b_per_w + g * L, L)] = v + j * V
            return 0

        lax.fori_loop(0, b_per_w // L, build_g, 0)
        pltpu.async_copy(t_hbm.at[gidx_v], buf_v, sem).wait()
        copies = [
            pltpu.async_copy(
                buf_v.at[pl.ds(j * b_per_w, b_per_w)],
                out_hbm.at[j, pl.ds(base, b_per_w)],
                sem2,
            )
            for j in range(D)
        ]
        for c in copies:
            c.wait()

    return k


def kernel(table, subject_ids):
    V, D = table.shape
    (B,) = subject_ids.shape
    tflat = table.T.reshape(-1)
    outT = _build(V, D, B)(tflat, subject_ids.astype(jnp.int32))
    return outT.T


# speedup vs baseline: 1.0013x; 1.0013x over previous
import functools

import jax
import jax.numpy as jnp
from jax import lax
from jax.experimental import pallas as pl
from jax.experimental.pallas import tpu as pltpu, tpu_sc as plsc


@functools.cache
def _build(V, D, B):
    info = plsc.get_sparse_core_info()
    NC, NS, L = info.num_cores, info.num_subcores, info.num_lanes
    NW = NC * NS
    assert B % (8 * NW) == 0 and D % L == 0
    b_per_w = B // NW
    n_el = b_per_w * D
    mesh = plsc.VectorSubcoreMesh(core_axis_name="c", subcore_axis_name="s")

    @functools.partial(
        pl.kernel,
        mesh=mesh,
        out_type=jax.ShapeDtypeStruct((D, B), jnp.float32),
        scratch_types=[
            pltpu.VMEM((b_per_w,), jnp.int32),
            pltpu.VMEM((n_el,), jnp.int32),
            pltpu.VMEM((n_el,), jnp.float32),
            pltpu.SemaphoreType.DMA,
            pltpu.SemaphoreType.DMA,
        ],
    )
    def k(t_hbm, idx_hbm, out_hbm, idx_v, gidx_v, buf_v, sem, sem2):
        wid = lax.axis_index("s") * NC + lax.axis_index("c")
        base = wid * b_per_w
        pltpu.sync_copy(idx_hbm.at[pl.ds(base, b_per_w)], idx_v)

        def build_g(g, _):
            v = idx_v[pl.ds(g * L, L)]
            for j in range(D):
                gidx_v[pl.ds(j * b_per_w + g * L, L)] = v + j * V
            return 0

        lax.fori_loop(0, b_per_w // L, build_g, 0)
        pltpu.async_copy(t_hbm.at[gidx_v], buf_v, sem).wait()
        copies = [
            pltpu.async_copy(
                buf_v.at[pl.ds(j * b_per_w, b_per_w)],
                out_hbm.at[j, pl.ds(base, b_per_w)],
                sem2,
            )
            for j in range(D)
        ]
        for c in copies:
            c.wait()

    return k


def kernel(table, subject_ids):
    V, D = table.shape
    (B,) = subject_ids.shape
    tflat = table.T.reshape(-1)
    outT = _build(V, D, B)(tflat, subject_ids.astype(jnp.int32))
    return outT.T


# speedup vs baseline: 1.0072x; 1.0059x over previous
import functools

import jax
import jax.numpy as jnp
from jax import lax
from jax.experimental import pallas as pl
from jax.experimental.pallas import tpu as pltpu, tpu_sc as plsc


@functools.cache
def _build(V, D, B):
    info = plsc.get_sparse_core_info()
    NC, NS, L = info.num_cores, info.num_subcores, info.num_lanes
    NW = NC * NS
    assert B % (8 * NW) == 0 and D % L == 0
    b_per_w = B // NW
    n_el = b_per_w * D
    mesh = plsc.VectorSubcoreMesh(core_axis_name="c", subcore_axis_name="s")

    @functools.partial(
        pl.kernel,
        mesh=mesh,
        out_type=jax.ShapeDtypeStruct((D, B), jnp.float32),
        scratch_types=[
            pltpu.VMEM((b_per_w,), jnp.int32),
            pltpu.VMEM((n_el,), jnp.int32),
            pltpu.VMEM((n_el,), jnp.float32),
            pltpu.SemaphoreType.DMA,
            pltpu.SemaphoreType.DMA,
        ],
    )
    def k(t_hbm, idx_hbm, out_hbm, idx_v, gidx_v, buf_v, sem, sem2):
        wid = lax.axis_index("s") * NC + lax.axis_index("c")
        base = wid * b_per_w
        pltpu.sync_copy(idx_hbm.at[pl.ds(base, b_per_w)], idx_v)

        def build_g(g, _):
            v = idx_v[pl.ds(g * L, L)]
            for j in range(D):
                gidx_v[pl.ds(j * b_per_w + g * L, L)] = v + j * V
            return 0

        lax.fori_loop(0, b_per_w // L, build_g, 0)
        gathers = [
            pltpu.async_copy(
                t_hbm.at[gidx_v.at[pl.ds(j * b_per_w, b_per_w)]],
                buf_v.at[pl.ds(j * b_per_w, b_per_w)],
                sem,
            )
            for j in range(D)
        ]
        outs = []
        for j in range(D):
            gathers[j].wait()
            outs.append(
                pltpu.async_copy(
                    buf_v.at[pl.ds(j * b_per_w, b_per_w)],
                    out_hbm.at[j, pl.ds(base, b_per_w)],
                    sem2,
                )
            )
        for c in outs:
            c.wait()

    return k


def kernel(table, subject_ids):
    V, D = table.shape
    (B,) = subject_ids.shape
    tflat = table.T.reshape(-1)
    outT = _build(V, D, B)(tflat, subject_ids.astype(jnp.int32))
    return outT.T


# speedup vs baseline: 1.0220x; 1.0147x over previous
import functools

import jax
import jax.numpy as jnp
from jax import lax
from jax.experimental import pallas as pl
from jax.experimental.pallas import tpu as pltpu, tpu_sc as plsc


@functools.cache
def _build(V, D, B):
    info = plsc.get_sparse_core_info()
    NC, NS, L = info.num_cores, info.num_subcores, info.num_lanes
    NW = NC * NS
    assert B % (8 * NW) == 0 and D % L == 0
    b_per_w = B // NW
    n_el = b_per_w * D
    mesh = plsc.VectorSubcoreMesh(core_axis_name="c", subcore_axis_name="s")

    @functools.partial(
        pl.kernel,
        mesh=mesh,
        out_type=jax.ShapeDtypeStruct((D, B), jnp.float32),
        scratch_types=[
            pltpu.VMEM((b_per_w,), jnp.int32),
            pltpu.VMEM((n_el,), jnp.int32),
            pltpu.VMEM((n_el,), jnp.float32),
            pltpu.SemaphoreType.DMA,
            pltpu.SemaphoreType.DMA,
        ],
    )
    def k(t_hbm, idx_hbm, out_hbm, idx_v, gidx_v, buf_v, sem, sem2):
        wid = lax.axis_index("s") * NC + lax.axis_index("c")
        base = wid * b_per_w
        pltpu.sync_copy(idx_hbm.at[pl.ds(base, b_per_w)], idx_v)

        gathers = []
        for j in range(D):

            def build_g(g, _, j=j):
                v = idx_v[pl.ds(g * L, L)]
                gidx_v[pl.ds(j * b_per_w + g * L, L)] = v + j * V
                return 0

            lax.fori_loop(0, b_per_w // L, build_g, 0)
            gathers.append(
                pltpu.async_copy(
                    t_hbm.at[gidx_v.at[pl.ds(j * b_per_w, b_per_w)]],
                    buf_v.at[pl.ds(j * b_per_w, b_per_w)],
                    sem,
                )
            )
        outs = []
        for j in range(D):
            gathers[j].wait()
            outs.append(
                pltpu.async_copy(
                    buf_v.at[pl.ds(j * b_per_w, b_per_w)],
                    out_hbm.at[j, pl.ds(base, b_per_w)],
                    sem2,
                )
            )
        for c in outs:
            c.wait()

    return k


def kernel(table, subject_ids):
    V, D = table.shape
    (B,) = subject_ids.shape
    tflat = table.T.reshape(-1)
    outT = _build(V, D, B)(tflat, subject_ids.astype(jnp.int32))
    return outT.T
